# baseline (speedup 1.0000x reference)
import jax
import jax.numpy as jnp
from jax.experimental import pallas as pl
from jax.experimental.pallas import tpu as pltpu

TILE = 1024
NBUF = 3
NSPLIT = 4


def _router_kernel(x_hbm, w_ref, bias_ref, out_ref, xbuf, sems):
    n_tiles = pl.num_programs(0)
    i = pl.program_id(0)
    dim = x_hbm.shape[1]
    q = dim // NSPLIT

    def start(t, slot):
        for k in range(NSPLIT):
            pltpu.make_async_copy(
                x_hbm.at[pl.ds(t * TILE, TILE), pl.ds(k * q, q)],
                xbuf.at[slot, :, pl.ds(k * q, q)],
                sems.at[slot, k],
            ).start()

    def wait(t, slot):
        for k in range(NSPLIT):
            pltpu.make_async_copy(
                x_hbm.at[pl.ds(t * TILE, TILE), pl.ds(k * q, q)],
                xbuf.at[slot, :, pl.ds(k * q, q)],
                sems.at[slot, k],
            ).wait()

    @pl.when(i == 0)
    def _():
        for t in range(NBUF - 1):
            start(t, t)

    nxt = i + NBUF - 1
    @pl.when(nxt < n_tiles)
    def _():
        start(nxt, jax.lax.rem(nxt, NBUF))

    slot = jax.lax.rem(i, NBUF)
    wait(i, slot)

    logits = jax.lax.dot_general(
        xbuf[slot], w_ref[...],
        dimension_numbers=(((1,), (1,)), ((), ())),
        preferred_element_type=jnp.float32,
    )
    logits = logits + bias_ref[...]
    m = jnp.max(logits, axis=-1, keepdims=True)
    e = jnp.exp(logits - m)
    out_ref[...] = e / jnp.sum(e, axis=-1, keepdims=True)


@jax.jit
def kernel(x, W, bias):
    n_tokens, dim = x.shape
    n_experts = W.shape[0]
    grid = (n_tokens // TILE,)
    return pl.pallas_call(
        _router_kernel,
        grid=grid,
        in_specs=[
            pl.BlockSpec(memory_space=pltpu.MemorySpace.HBM),
            pl.BlockSpec((n_experts, dim), lambda i: (0, 0)),
            pl.BlockSpec((1, n_experts), lambda i: (0, 0)),
        ],
        out_specs=pl.BlockSpec((TILE, n_experts), lambda i: (i, 0)),
        out_shape=jax.ShapeDtypeStruct((n_tokens, n_experts), jnp.float32),
        scratch_shapes=[
            pltpu.VMEM((NBUF, TILE, dim), jnp.float32),
            pltpu.SemaphoreType.DMA((NBUF, NSPLIT)),
        ],
        compiler_params=pltpu.CompilerParams(
            dimension_semantics=("arbitrary",),
        ),
    )(x, W, bias.reshape(1, n_experts))
